# baseline (speedup 1.0000x reference)
import jax
import jax.numpy as jnp
import numpy as np
from jax.experimental import pallas as pl
from jax.experimental.pallas import tpu as pltpu

B, C, L, D, E, P = 32, 16, 16, 512, 8, 720
_EPS = float(np.finfo(np.float64).eps)


def _lph_kernel(*refs):
    xs_refs = refs[0:E]
    w_refs = refs[E:2 * E]
    b_ref = refs[2 * E]
    g_ref = refs[2 * E + 1]
    out_ref = refs[2 * E + 2]
    x_scr = refs[2 * E + 3]
    w_scr = refs[2 * E + 4]
    sem = refs[2 * E + 5]

    def x_copy(e):
        return pltpu.make_async_copy(
            xs_refs[e].at[:, :, pl.ds(L - 1, 1), :], x_scr.at[e], sem.at[2 * e]
        )

    def w_copy(e):
        return pltpu.make_async_copy(w_refs[e], w_scr.at[e], sem.at[2 * e + 1])

    for e in range(E):
        x_copy(e).start()
        w_copy(e).start()

    acc = None
    for e in range(E):
        x_copy(e).wait()
        w_copy(e).wait()
        x = x_scr[e].reshape(B * C, D)
        w = w_scr[e]
        yt = jax.lax.dot_general(
            w, x, (((1,), (1,)), ((), ())), preferred_element_type=jnp.float32
        )
        term = jnp.exp(yt + b_ref[e]) * g_ref[e]
        acc = term if acc is None else acc + term

    res = jnp.log(jnp.where(acc == 0.0, _EPS, acc))
    for b in range(B):
        out_ref[b] = res[:, b * C:(b + 1) * C]


@jax.jit
def kernel(xs0, W0, b0, xs1, W1, b1, xs2, W2, b2, xs3, W3, b3,
           xs4, W4, b4, xs5, W5, b5, xs6, W6, b6, xs7, W7, b7, gates):
    xs = [xs0, xs1, xs2, xs3, xs4, xs5, xs6, xs7]
    Ws = [W0, W1, W2, W3, W4, W5, W6, W7]
    bias = jnp.stack([b0, b1, b2, b3, b4, b5, b6, b7]).reshape(E, P, 1)
    g_lanes = jnp.repeat(gates, C, axis=0).T.reshape(E, 1, B * C)

    any_spec = pl.BlockSpec(memory_space=pltpu.MemorySpace.HBM)

    out = pl.pallas_call(
        _lph_kernel,
        in_specs=[any_spec] * (2 * E) + [
            pl.BlockSpec((E, P, 1), lambda: (0, 0, 0)),
            pl.BlockSpec((E, 1, B * C), lambda: (0, 0, 0)),
        ],
        out_specs=pl.BlockSpec((B, P, C), lambda: (0, 0, 0)),
        out_shape=jax.ShapeDtypeStruct((B, P, C), jnp.float32),
        scratch_shapes=[
            pltpu.VMEM((E, B, C, 1, D), jnp.float32),
            pltpu.VMEM((E, P, D), jnp.float32),
            pltpu.SemaphoreType.DMA((2 * E,)),
        ],
    )(*xs, *Ws, bias, g_lanes)
    return out


# speedup vs baseline: 1.4203x; 1.4203x over previous
import jax
import jax.numpy as jnp
import numpy as np
from jax.experimental import pallas as pl
from jax.experimental.pallas import tpu as pltpu

B, C, L, D, E, P = 32, 16, 16, 512, 8, 720
_EPS = float(np.finfo(np.float64).eps)


def _lph_kernel(*refs):
    xs_refs = refs[0:E]
    w_refs = refs[E:2 * E]
    b_refs = refs[2 * E:3 * E]
    g_ref = refs[3 * E]
    out_ref = refs[3 * E + 1]
    x_scr = refs[3 * E + 2]
    w_scr = refs[3 * E + 3]
    sem = refs[3 * E + 4]

    def x_copy(e):
        return pltpu.make_async_copy(
            xs_refs[e].at[:, :, pl.ds(L - 1, 1), :], x_scr.at[e], sem.at[2 * e]
        )

    def w_copy(e):
        return pltpu.make_async_copy(w_refs[e], w_scr.at[e], sem.at[2 * e + 1])

    for e in range(E):
        x_copy(e).start()
        w_copy(e).start()

    acc = None
    for e in range(E):
        x_copy(e).wait()
        w_copy(e).wait()
        x = x_scr[e].reshape(B * C, D)
        w = w_scr[e]
        y = jax.lax.dot_general(
            x, w, (((1,), (1,)), ((), ())), preferred_element_type=jnp.float32
        )
        term = jnp.exp(y + b_refs[e][...]) * g_ref[e]
        acc = term if acc is None else acc + term

    res = jnp.log(jnp.where(acc == 0.0, _EPS, acc))
    out_ref[...] = res.reshape(B, C, P)


@jax.jit
def kernel(xs0, W0, b0, xs1, W1, b1, xs2, W2, b2, xs3, W3, b3,
           xs4, W4, b4, xs5, W5, b5, xs6, W6, b6, xs7, W7, b7, gates):
    xs = [xs0, xs1, xs2, xs3, xs4, xs5, xs6, xs7]
    Ws = [W0, W1, W2, W3, W4, W5, W6, W7]
    bs = [b.reshape(1, P) for b in [b0, b1, b2, b3, b4, b5, b6, b7]]
    g_rows = jnp.repeat(gates, C, axis=0).T.reshape(E, B * C, 1)

    any_spec = pl.BlockSpec(memory_space=pltpu.MemorySpace.HBM)

    out = pl.pallas_call(
        _lph_kernel,
        in_specs=[any_spec] * (2 * E) + [
            pl.BlockSpec((1, P), lambda: (0, 0)) for _ in range(E)
        ] + [
            pl.BlockSpec((E, B * C, 1), lambda: (0, 0, 0)),
        ],
        out_specs=pl.BlockSpec((B, C, P), lambda: (0, 0, 0)),
        out_shape=jax.ShapeDtypeStruct((B, C, P), jnp.float32),
        scratch_shapes=[
            pltpu.VMEM((E, B, C, 1, D), jnp.float32),
            pltpu.VMEM((E, P, D), jnp.float32),
            pltpu.SemaphoreType.DMA((2 * E,)),
        ],
    )(*xs, *Ws, *bs, g_rows)
    return jnp.transpose(out, (0, 2, 1))
